# baseline (speedup 1.0000x reference)
import functools

import jax
import jax.numpy as jnp
from jax.experimental import pallas as pl
from jax.experimental.pallas import tpu as pltpu

B, V, D, C = 16384, 1000000, 64, 128
BLK = 1024


def _compute_body(emb_ref, ctx_ref, wc_ref, bc_ref, wd_ref, bd_ref, out_ref):
    emb = emb_ref[...]
    ctx = ctx_ref[...]
    ctx_t = jax.nn.sigmoid(
        jnp.dot(ctx, wc_ref[...], preferred_element_type=jnp.float32) + bc_ref[...]
    )
    pre = jnp.dot(emb, wd_ref[...], preferred_element_type=jnp.float32) + bd_ref[...]
    delta = jax.nn.sigmoid(pre * ctx_t)
    upd = delta * emb + (1.0 - delta) * ctx_t
    denom = jnp.maximum(jnp.sqrt(jnp.sum(upd * upd, axis=-1, keepdims=True)), 1e-12)
    out_ref[...] = upd / denom


def _tc_compute(emb, context, W_ctx, b_ctx, W_delta, b_delta):
    grid = (B // BLK,)
    return pl.pallas_call(
        _compute_body,
        grid=grid,
        in_specs=[
            pl.BlockSpec((BLK, D), lambda i: (i, 0)),
            pl.BlockSpec((BLK, C), lambda i: (i, 0)),
            pl.BlockSpec((C, D), lambda i: (0, 0)),
            pl.BlockSpec((1, D), lambda i: (0, 0)),
            pl.BlockSpec((D, D), lambda i: (0, 0)),
            pl.BlockSpec((1, D), lambda i: (0, 0)),
        ],
        out_specs=pl.BlockSpec((BLK, D), lambda i: (i, 0)),
        out_shape=jax.ShapeDtypeStruct((B, D), jnp.float32),
    )(emb, context, W_ctx, b_ctx.reshape(1, D), W_delta, b_delta.reshape(1, D))


def kernel(inputs, context, table, W_ctx, b_ctx, W_delta, b_delta):
    idx = inputs.reshape(B).astype(jnp.int32)
    emb = jnp.take(table, idx, axis=0)
    out = _tc_compute(emb, context, W_ctx, b_ctx, W_delta, b_delta)
    aux = jnp.zeros((V,), jnp.int32).at[idx].max(jnp.arange(B, dtype=jnp.int32))
    w = aux[idx]
    vals = jnp.take(out, w, axis=0)
    table_new = table.at[idx].set(vals)
    return out, table_new
